# baseline (speedup 1.0000x reference)
import numpy as np
import jax
import jax.numpy as jnp
from jax.experimental import pallas as pl
from jax.experimental.pallas import tpu as pltpu


def _band0(conv0):
    d = np.arange(5)[:, None, None, None]
    i = np.arange(16)[None, :, None, None]
    P = np.arange(2)[None, None, :, None]
    p = np.arange(6)[None, None, None, :]
    M0 = jnp.asarray((i == 2 * p + P + d).astype(np.float32))
    j = np.arange(6)[:, None, None]
    D = np.arange(2)[None, :, None]
    r = np.arange(5)[None, None, :]
    R = jnp.asarray((j - D == r).astype(np.float32))
    A0 = jnp.einsum('rdc,diPp,jDr->jiDPpc', conv0[:, :, 0, :], M0, R)
    return A0.reshape(96, 768)


def _band1(conv1):
    d = np.arange(5)[:, None, None, None]
    w = np.arange(8)[None, :, None, None]
    P = np.arange(2)[None, None, :, None]
    q = np.arange(2)[None, None, None, :]
    M1 = jnp.asarray((w == 2 * q + P + d).astype(np.float32))
    A1 = jnp.einsum('rdio,dwPq->rwiPqo', conv1, M1).reshape(5, 256, 256)
    z = jnp.zeros((1, 256, 256), A1.dtype)
    return jnp.concatenate(
        [jnp.concatenate([A1, z], 0), jnp.concatenate([z, A1], 0)], axis=2)


def _fused_kernel(x_ref, a0_ref, a1_ref, w1_ref, w2_ref, o_ref, f1_ref, f2_ref):
    f32 = jnp.float32
    a0 = a0_ref[...]
    a1 = a1_ref[...]
    for ph in range(12):
        for h in range(2):
            base = h * 448 + 32 * ph
            o = jnp.dot(x_ref[:, base: base + 96], a0,
                        preferred_element_type=f32).astype(f1_ref.dtype)
            v = jnp.maximum(jnp.maximum(o[:, :384], o[:, 384:]), 0.0)
            f1_ref[:, ph * 384 + h * 192: ph * 384 + (h + 1) * 192] = \
                jnp.maximum(v[:, :192], v[:, 192:])
    for ph in range(4):
        for h in range(2):
            off = 2 * ph * 384 + h * 128
            acc = jnp.dot(f1_ref[:, off: off + 256], a1[0],
                          preferred_element_type=f32)
            for j in range(1, 6):
                acc = acc + jnp.dot(
                    f1_ref[:, off + j * 384: off + j * 384 + 256], a1[j],
                    preferred_element_type=f32)
            acc = acc.astype(f2_ref.dtype)
            v = jnp.maximum(jnp.maximum(acc[:, :256], acc[:, 256:]), 0.0)
            f2_ref[:, ph * 256 + h * 128: ph * 256 + (h + 1) * 128] = jnp.maximum(
                v[:, :128], v[:, 128:])
    h = jnp.maximum(jnp.dot(f2_ref[...], w1_ref[...], preferred_element_type=f32), 0.0)
    y = jnp.dot(h.astype(w2_ref.dtype), w2_ref[...], preferred_element_type=f32)
    m = jnp.max(y, axis=-1, keepdims=True)
    s = y - m
    lse = jnp.log(jnp.sum(jnp.exp(s), axis=-1, keepdims=True))
    o_ref[...] = (s - lse).astype(o_ref.dtype)


def kernel(x, conv0, conv1, fc0, fc1):
    B = x.shape[0]
    T = 1024 if B % 1024 == 0 else (128 if B % 128 == 0 else B)
    n_hid = fc0.shape[1]
    cd = jnp.bfloat16
    x3 = x.reshape(B, 28, 28).astype(cd)
    x2 = jnp.concatenate(
        [x3[:, :, 0:16].reshape(B, 448), x3[:, :, 12:28].reshape(B, 448)], axis=1)
    A0 = _band0(conv0).astype(cd)
    A1 = _band1(conv1).astype(cd)
    W1 = fc0.reshape(64, 4, 4, n_hid).transpose(1, 2, 0, 3).reshape(1024, n_hid).astype(cd)
    return pl.pallas_call(
        _fused_kernel,
        out_shape=jax.ShapeDtypeStruct((B, 10), x.dtype),
        grid=(B // T,),
        in_specs=[
            pl.BlockSpec((T, 896), lambda i: (i, 0)),
            pl.BlockSpec((96, 768), lambda i: (0, 0)),
            pl.BlockSpec((6, 256, 512), lambda i: (0, 0, 0)),
            pl.BlockSpec((1024, n_hid), lambda i: (0, 0)),
            pl.BlockSpec((n_hid, 10), lambda i: (0, 0)),
        ],
        out_specs=pl.BlockSpec((T, 10), lambda i: (i, 0)),
        scratch_shapes=[
            pltpu.VMEM((T, 4608), cd),
            pltpu.VMEM((T, 1024), cd),
        ],
        compiler_params=pltpu.CompilerParams(dimension_semantics=("parallel",)),
    )(x2, A0, A1, W1, fc1.astype(cd))


# speedup vs baseline: 1.0710x; 1.0710x over previous
import numpy as np
import jax
import jax.numpy as jnp
from jax.experimental import pallas as pl
from jax.experimental.pallas import tpu as pltpu


def _band0(conv0):
    d = np.arange(5)[:, None, None, None]
    i = np.arange(16)[None, :, None, None]
    P = np.arange(2)[None, None, :, None]
    p = np.arange(6)[None, None, None, :]
    M0 = jnp.asarray((i == 2 * p + P + d).astype(np.float32))
    j = np.arange(6)[:, None, None]
    D = np.arange(2)[None, :, None]
    r = np.arange(5)[None, None, :]
    R = jnp.asarray((j - D == r).astype(np.float32))
    A0 = jnp.einsum('rdc,diPp,jDr->jiDPpc', conv0[:, :, 0, :], M0, R)
    return A0.reshape(96, 768)


def _band1(conv1):
    d = np.arange(5)[:, None, None, None]
    w = np.arange(8)[None, :, None, None]
    P = np.arange(2)[None, None, :, None]
    q = np.arange(2)[None, None, None, :]
    M1 = jnp.asarray((w == 2 * q + P + d).astype(np.float32))
    A1 = jnp.einsum('rdio,dwPq->rwiPqo', conv1, M1)
    return A1.reshape(5, 256, 256)


def _fused_kernel(x_ref, a0_ref, a1_ref, w1_ref, w2_ref, o_ref, f1_ref, f2_ref):
    f32 = jnp.float32
    a0 = a0_ref[...]
    a1 = a1_ref[...]
    for ph in range(12):
        for h in range(2):
            base = h * 448 + 32 * ph
            o = jnp.dot(x_ref[:, base: base + 96], a0,
                        preferred_element_type=f32).astype(f1_ref.dtype)
            v = jnp.maximum(jnp.maximum(o[:, :384], o[:, 384:]), 0.0)
            f1_ref[:, ph * 384 + h * 192: ph * 384 + (h + 1) * 192] = \
                jnp.maximum(v[:, :192], v[:, 192:])
    for ph in range(4):
        for h in range(2):
            zz = []
            for dh in range(2):
                off = (2 * ph + dh) * 384 + h * 128
                acc = jnp.dot(f1_ref[:, off: off + 256], a1[0],
                              preferred_element_type=f32)
                for r in range(1, 5):
                    acc = acc + jnp.dot(
                        f1_ref[:, off + r * 384: off + r * 384 + 256], a1[r],
                        preferred_element_type=f32)
                zz.append(acc.astype(f2_ref.dtype))
            v = jnp.maximum(jnp.maximum(zz[0], zz[1]), 0.0)
            f2_ref[:, ph * 256 + h * 128: ph * 256 + (h + 1) * 128] = jnp.maximum(
                v[:, :128], v[:, 128:])
    h = jnp.maximum(jnp.dot(f2_ref[...], w1_ref[...], preferred_element_type=f32), 0.0)
    y = jnp.dot(h.astype(w2_ref.dtype), w2_ref[...], preferred_element_type=f32)
    m = jnp.max(y, axis=-1, keepdims=True)
    s = y - m
    lse = jnp.log(jnp.sum(jnp.exp(s), axis=-1, keepdims=True))
    o_ref[...] = (s - lse).astype(o_ref.dtype)


def kernel(x, conv0, conv1, fc0, fc1):
    B = x.shape[0]
    T = 1024 if B % 1024 == 0 else (128 if B % 128 == 0 else B)
    n_hid = fc0.shape[1]
    cd = jnp.bfloat16
    x3 = x.reshape(B, 28, 28).astype(cd)
    x2 = jnp.concatenate(
        [x3[:, :, 0:16].reshape(B, 448), x3[:, :, 12:28].reshape(B, 448)], axis=1)
    A0 = _band0(conv0).astype(cd)
    A1 = _band1(conv1).astype(cd)
    W1 = fc0.reshape(64, 4, 4, n_hid).transpose(1, 2, 0, 3).reshape(1024, n_hid).astype(cd)
    return pl.pallas_call(
        _fused_kernel,
        out_shape=jax.ShapeDtypeStruct((B, 10), x.dtype),
        grid=(B // T,),
        in_specs=[
            pl.BlockSpec((T, 896), lambda i: (i, 0)),
            pl.BlockSpec((96, 768), lambda i: (0, 0)),
            pl.BlockSpec((5, 256, 256), lambda i: (0, 0, 0)),
            pl.BlockSpec((1024, n_hid), lambda i: (0, 0)),
            pl.BlockSpec((n_hid, 10), lambda i: (0, 0)),
        ],
        out_specs=pl.BlockSpec((T, 10), lambda i: (i, 0)),
        scratch_shapes=[
            pltpu.VMEM((T, 4608), cd),
            pltpu.VMEM((T, 1024), cd),
        ],
        compiler_params=pltpu.CompilerParams(dimension_semantics=("parallel",)),
    )(x2, A0, A1, W1, fc1.astype(cd))
